# Initial kernel scaffold; baseline (speedup 1.0000x reference)
import functools

import jax
import jax.numpy as jnp
from jax import lax
from jax.experimental import pallas as pl
from jax.experimental.pallas import tpu as pltpu
from jax.experimental.pallas import tpu_sc as plsc

_N = 10000
_E = 320000
_BASIS = 128
_NG = 16
_HID = 256
_NGRAPHS = 64
_T = 3

_NC = 2
_NS = 16
_NW = _NC * _NS

_CH = 80


def _make_sc_gather(n_idx, group):
    per_w = n_idx // _NW
    gsz = _CH * group
    n_groups = per_w // gsz
    assert per_w % gsz == 0
    mesh = plsc.VectorSubcoreMesh(core_axis_name="c", subcore_axis_name="s")

    def body(table_hbm, idx_hbm, out_hbm, idx_v, rows_v, sem):
        wid = lax.axis_index("s") * _NC + lax.axis_index("c")
        base0 = wid * per_w

        def group_body(g, _):
            base = base0 + g * gsz
            pltpu.sync_copy(idx_hbm.at[pl.ds(base, gsz)], idx_v)
            copies = [
                pltpu.async_copy(
                    table_hbm.at[idx_v.at[pl.ds(k * _CH, _CH)]],
                    rows_v.at[pl.ds(k * _CH, _CH)],
                    sem,
                )
                for k in range(group)
            ]
            for c in copies:
                c.wait()
            pltpu.sync_copy(rows_v, out_hbm.at[pl.ds(base, gsz)])
            return 0

        lax.fori_loop(0, n_groups, group_body, 0)

    return functools.partial(
        pl.kernel,
        out_type=jax.ShapeDtypeStruct((n_idx, _BASIS), jnp.float32),
        mesh=mesh,
        scratch_types=[
            pltpu.VMEM((gsz,), jnp.int32),
            pltpu.VMEM((gsz, _BASIS), jnp.float32),
            pltpu.SemaphoreType.DMA,
        ],
    )(body)


def _make_sc_scatter(group):
    per_w = _E // _NW
    gsz = _CH * group
    n_groups = per_w // gsz
    assert per_w % gsz == 0
    rows_per_tile = _N // _NS
    zcopy = 125
    mesh = plsc.VectorSubcoreMesh(core_axis_name="c", subcore_axis_name="s")

    def body(m_hbm, dst2_hbm, out_hbm, idx_v, rows_v, acc_sh, sem):
        cid = lax.axis_index("c")
        sid = lax.axis_index("s")
        wid = sid * _NC + cid

        zeros16 = jnp.zeros((16,), jnp.float32)

        def zrow(r, _):
            for cc in range(_BASIS // 16):
                rows_v[r, pl.ds(cc * 16, 16)] = zeros16
            return 0

        lax.fori_loop(0, zcopy, zrow, 0)
        for k in range(rows_per_tile // zcopy):
            pltpu.sync_copy(
                rows_v.at[pl.ds(0, zcopy)],
                acc_sh.at[pl.ds(sid * rows_per_tile + k * zcopy, zcopy)],
            )
        plsc.subcore_barrier()

        def group_body(g, _):
            base = wid * per_w + g * gsz
            brow = base // _CH
            pltpu.sync_copy(m_hbm.at[pl.ds(base, gsz)], rows_v)
            pltpu.sync_copy(dst2_hbm.at[pl.ds(brow, group)], idx_v)
            for k in range(group):
                pltpu.sync_copy(
                    rows_v.at[pl.ds(k * _CH, _CH)],
                    acc_sh.at[idx_v.at[k]],
                    add=True,
                )
            return 0

        lax.fori_loop(0, n_groups, group_body, 0)
        plsc.subcore_barrier()
        pltpu.sync_copy(
            acc_sh.at[pl.ds(sid * rows_per_tile, rows_per_tile)],
            out_hbm.at[pl.ds(cid * _N + sid * rows_per_tile, rows_per_tile)],
        )

    return functools.partial(
        pl.kernel,
        out_type=jax.ShapeDtypeStruct((_NC * _N, _BASIS), jnp.float32),
        mesh=mesh,
        scratch_types=[
            pltpu.VMEM((group, _CH), jnp.int32),
            pltpu.VMEM((gsz, _BASIS), jnp.float32),
            pltpu.VMEM_SHARED((_N, _BASIS), jnp.float32),
            pltpu.SemaphoreType.DMA,
        ],
    )(body)


_sc_gather_nodes = _make_sc_gather(10240, group=4)
_sc_gather_edges = _make_sc_gather(_E, group=5)
_sc_scatter = _make_sc_scatter(group=5)


_BN = 2000
_BE = 2000


def _full(shape):
    return pl.BlockSpec(shape, lambda i: (0,) * len(shape))


def _ccf_first_body(c_ref, w_ref, b_ref, ccf_ref):
    ccf_ref[...] = (
        jnp.dot(c_ref[...], w_ref[...], preferred_element_type=jnp.float32)
        + b_ref[...]
    )


def _ccf_first(C, cf_w, cf_b):
    return pl.pallas_call(
        _ccf_first_body,
        grid=(_N // _BN,),
        in_specs=[
            pl.BlockSpec((_BN, _BASIS), lambda i: (i, 0)),
            _full((_BASIS, _BASIS)),
            _full((1, _BASIS)),
        ],
        out_specs=pl.BlockSpec((_BN, _BASIS), lambda i: (i, 0)),
        out_shape=jax.ShapeDtypeStruct((_N, _BASIS), jnp.float32),
    )(C, cf_w, cf_b.reshape(1, _BASIS))


def _ccf_step_body(c_ref, a0_ref, a1_ref, w_ref, b_ref, cn_ref, ccf_ref):
    c = c_ref[...] + a0_ref[...] + a1_ref[...]
    cn_ref[...] = c
    ccf_ref[...] = (
        jnp.dot(c, w_ref[...], preferred_element_type=jnp.float32) + b_ref[...]
    )


def _ccf_step(C, agg2, cf_w, cf_b):
    blk = pl.BlockSpec((_BN, _BASIS), lambda i: (i, 0))
    nb = _N // _BN
    return pl.pallas_call(
        _ccf_step_body,
        grid=(nb,),
        in_specs=[
            blk,
            pl.BlockSpec((_BN, _BASIS), lambda i: (i, 0)),
            pl.BlockSpec((_BN, _BASIS), lambda i: (i + nb, 0)),
            _full((_BASIS, _BASIS)),
            _full((1, _BASIS)),
        ],
        out_specs=[blk, blk],
        out_shape=[
            jax.ShapeDtypeStruct((_N, _BASIS), jnp.float32),
            jax.ShapeDtypeStruct((_N, _BASIS), jnp.float32),
        ],
    )(C, agg2, agg2, cf_w, cf_b.reshape(1, _BASIS))


def _dfe_body(e_ref, w_ref, b_ref, o_ref):
    o_ref[...] = (
        jnp.dot(e_ref[...], w_ref[...], preferred_element_type=jnp.float32)
        + b_ref[...]
    )


def _dfe(edge_attr, df_w, df_b):
    return pl.pallas_call(
        _dfe_body,
        grid=(_E // _BE,),
        in_specs=[
            pl.BlockSpec((_BE, _NG), lambda i: (i, 0)),
            _full((_NG, _BASIS)),
            _full((1, _BASIS)),
        ],
        out_specs=pl.BlockSpec((_BE, _BASIS), lambda i: (i, 0)),
        out_shape=jax.ShapeDtypeStruct((_E, _BASIS), jnp.float32),
    )(edge_attr, df_w, df_b.reshape(1, _BASIS))


def _edge_mm_body(g_ref, d_ref, w_ref, m_ref):
    p = g_ref[...] * d_ref[...]
    m_ref[...] = jnp.tanh(
        jnp.dot(p, w_ref[...], preferred_element_type=jnp.float32)
    )


def _edge_mm(G, dfe, fc_w):
    blk = pl.BlockSpec((_BE, _BASIS), lambda i: (i, 0))
    return pl.pallas_call(
        _edge_mm_body,
        grid=(_E // _BE,),
        in_specs=[blk, blk, _full((_BASIS, _BASIS))],
        out_specs=blk,
        out_shape=jax.ShapeDtypeStruct((_E, _BASIS), jnp.float32),
    )(G, dfe, fc_w)


def _readout_body(c_ref, a0_ref, a1_ref, b_ref, w1_ref, b1_ref, w2_ref,
                  b2_ref, o_ref):
    i = pl.program_id(0)
    c = c_ref[...] + a0_ref[...] + a1_ref[...]
    h1 = jnp.tanh(
        jnp.dot(c, w1_ref[...], preferred_element_type=jnp.float32)
        + b1_ref[...]
    )
    h = jnp.dot(h1, w2_ref[...], preferred_element_type=jnp.float32) + b2_ref[...]
    ids = b_ref[0, 0, :]
    onehot = (
        ids[:, None] == lax.broadcasted_iota(jnp.int32, (_BN, _NGRAPHS), 1)
    ).astype(jnp.float32)
    pooled = lax.dot_general(
        onehot, h, (((0,), (0,)), ((), ())),
        preferred_element_type=jnp.float32,
    )

    @pl.when(i == 0)
    def _():
        o_ref[...] = jnp.zeros_like(o_ref)

    o_ref[...] += pooled


def _readout(C, agg2, batch, mlp_w1, mlp_b1, mlp_w2, mlp_b2):
    nb = _N // _BN
    blk = pl.BlockSpec((_BN, _BASIS), lambda i: (i, 0))
    return pl.pallas_call(
        _readout_body,
        grid=(nb,),
        in_specs=[
            blk,
            pl.BlockSpec((_BN, _BASIS), lambda i: (i, 0)),
            pl.BlockSpec((_BN, _BASIS), lambda i: (i + nb, 0)),
            pl.BlockSpec((1, 1, _BN), lambda i: (i, 0, 0)),
            _full((_BASIS, _HID)),
            _full((1, _HID)),
            _full((_HID, 4)),
            _full((1, 4)),
        ],
        out_specs=pl.BlockSpec((_NGRAPHS, 4), lambda i: (0, 0)),
        out_shape=jax.ShapeDtypeStruct((_NGRAPHS, 4), jnp.float32),
    )(C, agg2, agg2, batch.reshape(nb, 1, _BN), mlp_w1,
      mlp_b1.reshape(1, _HID), mlp_w2, mlp_b2.reshape(1, 4))


def kernel(Z, edge_index, edge_attr, batch, embed, cf_w, cf_b, df_w, df_b,
           fc_w, mlp_w1, mlp_b1, mlp_w2, mlp_b2):
    src = edge_index[0].astype(jnp.int32)
    dst = edge_index[1].astype(jnp.int32)
    Zp = jnp.concatenate([Z.astype(jnp.int32), jnp.zeros((240,), jnp.int32)])
    dst2 = dst.reshape(_E // _CH, _CH)

    C = _sc_gather_nodes(embed, Zp)[:_N]
    dfe = _dfe(edge_attr, df_w, df_b)

    agg2 = None
    for t in range(_T):
        if t == 0:
            Ccf = _ccf_first(C, cf_w, cf_b)
        else:
            C, Ccf = _ccf_step(C, agg2, cf_w, cf_b)
        G = _sc_gather_edges(Ccf, src)
        M = _edge_mm(G, dfe, fc_w)
        agg2 = _sc_scatter(M, dst2)

    return _readout(C, agg2, batch, mlp_w1, mlp_b1, mlp_w2, mlp_b2)


# speedup vs baseline: 2.7727x; 2.7727x over previous
import functools

import jax
import jax.numpy as jnp
from jax import lax
from jax.experimental import pallas as pl
from jax.experimental.pallas import tpu as pltpu
from jax.experimental.pallas import tpu_sc as plsc

_N = 10000
_E = 320000
_BASIS = 128
_NG = 16
_HID = 256
_NGRAPHS = 64
_T = 3
_NPAD = 10240

_NC = 2
_NS = 16
_NW = _NC * _NS

_CH = 80


def _make_sc_gather(n_idx, group):
    per_w = n_idx // _NW
    gsz = _CH * group
    n_groups = per_w // gsz
    assert per_w % gsz == 0
    mesh = plsc.VectorSubcoreMesh(core_axis_name="c", subcore_axis_name="s", num_cores=_NC, num_subcores=_NS)

    def body(table_hbm, idx_hbm, out_hbm, idx_v, rows_v, sem):
        wid = lax.axis_index("s") * _NC + lax.axis_index("c")
        base0 = wid * per_w

        def group_body(g, _):
            base = base0 + g * gsz
            pltpu.sync_copy(idx_hbm.at[pl.ds(base, gsz)], idx_v)
            copies = [
                pltpu.async_copy(
                    table_hbm.at[idx_v.at[pl.ds(k * _CH, _CH)]],
                    rows_v.at[pl.ds(k * _CH, _CH)],
                    sem,
                )
                for k in range(group)
            ]
            for c in copies:
                c.wait()
            pltpu.sync_copy(rows_v, out_hbm.at[pl.ds(base, gsz)])
            return 0

        lax.fori_loop(0, n_groups, group_body, 0)

    return functools.partial(
        pl.kernel,
        out_type=jax.ShapeDtypeStruct((n_idx, _BASIS), jnp.float32),
        mesh=mesh,
        scratch_types=[
            pltpu.VMEM((gsz,), jnp.int32),
            pltpu.VMEM((gsz, _BASIS), jnp.float32),
            pltpu.SemaphoreType.DMA,
        ],
    )(body)


def _make_sc_scatter(group):
    per_w = _E // _NW
    gsz = _CH * group
    n_groups = per_w // gsz
    assert per_w % gsz == 0
    rows_per_tile = _NPAD // _NS
    zcopy = 128
    mesh = plsc.VectorSubcoreMesh(core_axis_name="c", subcore_axis_name="s", num_cores=_NC, num_subcores=_NS)

    def body(m_hbm, dst3_hbm, out_hbm, idx_v, rows_v, acc_sh, sem, sem2):
        cid = lax.axis_index("c")
        sid = lax.axis_index("s")
        wid = sid * _NC + cid
        sems = (sem, sem2)

        zeros16 = jnp.zeros((16,), jnp.float32)

        def zrow(r, _):
            for cc in range(_BASIS // 16):
                rows_v[0, r, pl.ds(cc * 16, 16)] = zeros16
            return 0

        lax.fori_loop(0, _CH, zrow, 0)
        for k in range(rows_per_tile // _CH):
            pltpu.sync_copy(
                rows_v.at[0],
                acc_sh.at[pl.ds(sid * rows_per_tile + k * _CH, _CH)],
            )
        plsc.subcore_barrier()

        def group_body(g, _):
            base = wid * per_w + g * gsz
            brow = base // gsz
            pltpu.sync_copy(dst3_hbm.at[brow], idx_v)
            cp0 = pltpu.async_copy(
                m_hbm.at[pl.ds(base, _CH)], rows_v.at[0], sems[0])
            copies = [cp0]
            for k in range(group):
                if k + 1 < group:
                    copies.append(pltpu.async_copy(
                        m_hbm.at[pl.ds(base + (k + 1) * _CH, _CH)],
                        rows_v.at[(k + 1) % 2], sems[(k + 1) % 2]))
                copies[k].wait()
                pltpu.sync_copy(
                    rows_v.at[k % 2],
                    acc_sh.at[idx_v.at[k]],
                    add=True,
                )
            return 0

        lax.fori_loop(0, n_groups, group_body, 0)
        plsc.subcore_barrier()
        pltpu.sync_copy(
            acc_sh.at[pl.ds(sid * rows_per_tile, rows_per_tile)],
            out_hbm.at[pl.ds(cid * _NPAD + sid * rows_per_tile, rows_per_tile)],
        )

    return functools.partial(
        pl.kernel,
        out_type=jax.ShapeDtypeStruct((_NC * _NPAD, _BASIS), jnp.float32),
        mesh=mesh,
        scratch_types=[
            pltpu.VMEM((group, _CH), jnp.int32),
            pltpu.VMEM((2, _CH, _BASIS), jnp.float32),
            pltpu.VMEM_SHARED((_NPAD, _BASIS), jnp.float32),
            pltpu.SemaphoreType.DMA,
            pltpu.SemaphoreType.DMA,
        ],
    )(body)


_sc_gather_nodes = _make_sc_gather(10240, group=4)
_sc_gather_edges = _make_sc_gather(_E, group=5)
_sc_scatter = _make_sc_scatter(group=5)


_BN = 2000
_BE = 2000


def _full(shape):
    return pl.BlockSpec(shape, lambda i: (0,) * len(shape))


def _ccf_first_body(c_ref, w_ref, b_ref, ccf_ref):
    ccf_ref[...] = (
        jnp.dot(c_ref[...], w_ref[...], preferred_element_type=jnp.float32)
        + b_ref[...]
    )


def _ccf_first(C, cf_w, cf_b):
    return pl.pallas_call(
        _ccf_first_body,
        grid=(_N // _BN,),
        in_specs=[
            pl.BlockSpec((_BN, _BASIS), lambda i: (i, 0)),
            _full((_BASIS, _BASIS)),
            _full((1, _BASIS)),
        ],
        out_specs=pl.BlockSpec((_BN, _BASIS), lambda i: (i, 0)),
        out_shape=jax.ShapeDtypeStruct((_N, _BASIS), jnp.float32),
    )(C, cf_w, cf_b.reshape(1, _BASIS))


def _ccf_step_body(c_ref, a0_ref, a1_ref, w_ref, b_ref, cn_ref, ccf_ref):
    c = c_ref[...] + a0_ref[...] + a1_ref[...]
    cn_ref[...] = c
    ccf_ref[...] = (
        jnp.dot(c, w_ref[...], preferred_element_type=jnp.float32) + b_ref[...]
    )


def _ccf_step(C, a0, a1, cf_w, cf_b):
    blk = pl.BlockSpec((_BN, _BASIS), lambda i: (i, 0))
    nb = _N // _BN
    return pl.pallas_call(
        _ccf_step_body,
        grid=(nb,),
        in_specs=[
            blk,
            blk,
            blk,
            _full((_BASIS, _BASIS)),
            _full((1, _BASIS)),
        ],
        out_specs=[blk, blk],
        out_shape=[
            jax.ShapeDtypeStruct((_N, _BASIS), jnp.float32),
            jax.ShapeDtypeStruct((_N, _BASIS), jnp.float32),
        ],
    )(C, a0, a1, cf_w, cf_b.reshape(1, _BASIS))


def _dfe_body(e_ref, w_ref, b_ref, o_ref):
    o_ref[...] = (
        jnp.dot(e_ref[...], w_ref[...], preferred_element_type=jnp.float32)
        + b_ref[...]
    )


def _dfe(edge_attr, df_w, df_b):
    return pl.pallas_call(
        _dfe_body,
        grid=(_E // _BE,),
        in_specs=[
            pl.BlockSpec((_BE, _NG), lambda i: (i, 0)),
            _full((_NG, _BASIS)),
            _full((1, _BASIS)),
        ],
        out_specs=pl.BlockSpec((_BE, _BASIS), lambda i: (i, 0)),
        out_shape=jax.ShapeDtypeStruct((_E, _BASIS), jnp.float32),
    )(edge_attr, df_w, df_b.reshape(1, _BASIS))


def _edge_mm_body(g_ref, d_ref, w_ref, m_ref):
    p = g_ref[...] * d_ref[...]
    m_ref[...] = jnp.tanh(
        jnp.dot(p, w_ref[...], preferred_element_type=jnp.float32)
    )


def _edge_mm(G, dfe, fc_w):
    blk = pl.BlockSpec((_BE, _BASIS), lambda i: (i, 0))
    return pl.pallas_call(
        _edge_mm_body,
        grid=(_E // _BE,),
        in_specs=[blk, blk, _full((_BASIS, _BASIS))],
        out_specs=blk,
        out_shape=jax.ShapeDtypeStruct((_E, _BASIS), jnp.float32),
    )(G, dfe, fc_w)


def _readout_body(c_ref, a0_ref, a1_ref, b_ref, w1_ref, b1_ref, w2_ref,
                  b2_ref, o_ref):
    i = pl.program_id(0)
    c = c_ref[...] + a0_ref[...] + a1_ref[...]
    h1 = jnp.tanh(
        jnp.dot(c, w1_ref[...], preferred_element_type=jnp.float32)
        + b1_ref[...]
    )
    h = jnp.dot(h1, w2_ref[...], preferred_element_type=jnp.float32) + b2_ref[...]
    ids = b_ref[0, 0, :]
    onehot = (
        ids[:, None] == lax.broadcasted_iota(jnp.int32, (_BN, _NGRAPHS), 1)
    ).astype(jnp.float32)
    pooled = lax.dot_general(
        onehot, h, (((0,), (0,)), ((), ())),
        preferred_element_type=jnp.float32,
    )

    @pl.when(i == 0)
    def _():
        o_ref[...] = jnp.zeros_like(o_ref)

    o_ref[...] += pooled


def _readout(C, a0, a1, batch, mlp_w1, mlp_b1, mlp_w2, mlp_b2):
    nb = _N // _BN
    blk = pl.BlockSpec((_BN, _BASIS), lambda i: (i, 0))
    return pl.pallas_call(
        _readout_body,
        grid=(nb,),
        in_specs=[
            blk,
            blk,
            blk,
            pl.BlockSpec((1, 1, _BN), lambda i: (i, 0, 0)),
            _full((_BASIS, _HID)),
            _full((1, _HID)),
            _full((_HID, 4)),
            _full((1, 4)),
        ],
        out_specs=pl.BlockSpec((_NGRAPHS, 4), lambda i: (0, 0)),
        out_shape=jax.ShapeDtypeStruct((_NGRAPHS, 4), jnp.float32),
    )(C, a0, a1, batch.reshape(nb, 1, _BN), mlp_w1,
      mlp_b1.reshape(1, _HID), mlp_w2, mlp_b2.reshape(1, 4))


def kernel(Z, edge_index, edge_attr, batch, embed, cf_w, cf_b, df_w, df_b,
           fc_w, mlp_w1, mlp_b1, mlp_w2, mlp_b2):
    src = edge_index[0].astype(jnp.int32)
    dst = edge_index[1].astype(jnp.int32)
    Zp = jnp.concatenate([Z.astype(jnp.int32), jnp.zeros((240,), jnp.int32)])
    dst3 = dst.reshape(-1, 5, _CH)

    C = _sc_gather_nodes(embed, Zp)[:_N]
    dfe = _dfe(edge_attr, df_w, df_b)

    a0 = a1 = None
    for t in range(_T):
        if t == 0:
            Ccf = _ccf_first(C, cf_w, cf_b)
        else:
            C, Ccf = _ccf_step(C, a0, a1, cf_w, cf_b)
        G = _sc_gather_edges(Ccf, src)
        M = _edge_mm(G, dfe, fc_w)
        agg = _sc_scatter(M, dst3)
        a0 = agg[:_N]
        a1 = agg[_NPAD:_NPAD + _N]

    return _readout(C, a0, a1, batch, mlp_w1, mlp_b1, mlp_w2, mlp_b2)


# speedup vs baseline: 2.8160x; 1.0156x over previous
import functools

import jax
import jax.numpy as jnp
from jax import lax
from jax.experimental import pallas as pl
from jax.experimental.pallas import tpu as pltpu
from jax.experimental.pallas import tpu_sc as plsc

_N = 10000
_E = 320000
_BASIS = 128
_NG = 16
_HID = 256
_NGRAPHS = 64
_T = 3
_NPAD = 10240

_NC = 2
_NS = 16
_NW = _NC * _NS

_CH = 80


def _make_sc_gather(n_idx, ch, group):
    per_w = n_idx // _NW
    gsz = ch * group
    n_groups = per_w // gsz
    assert per_w % gsz == 0
    mesh = plsc.VectorSubcoreMesh(core_axis_name="c", subcore_axis_name="s", num_cores=_NC, num_subcores=_NS)

    n_pairs = n_groups // 2
    leftover = n_groups % 2 == 1

    def body(table_hbm, idx_hbm, out_hbm, idx_full, rows0, rows1,
             gsem, ssem0, ssem1):
        wid = lax.axis_index("s") * _NC + lax.axis_index("c")
        base0 = wid * per_w
        pltpu.sync_copy(idx_hbm.at[pl.ds(base0, per_w)], idx_full)

        def drain_store(ssem):
            pltpu.make_async_copy(
                rows0, out_hbm.at[pl.ds(0, gsz)], ssem).wait()

        def do_group(g, rows, ssem):
            off = g * gsz
            copies = [
                pltpu.async_copy(
                    table_hbm.at[idx_full.at[pl.ds(off + k * ch, ch)]],
                    rows.at[pl.ds(k * ch, ch)],
                    gsem,
                )
                for k in range(group)
            ]
            for c in copies:
                c.wait()
            pltpu.async_copy(rows, out_hbm.at[pl.ds(base0 + off, gsz)], ssem)

        if n_pairs > 0:
            do_group(0, rows0, ssem0)
            do_group(1, rows1, ssem1)

            def pair_body(j, _):
                drain_store(ssem0)
                do_group(2 * j, rows0, ssem0)
                drain_store(ssem1)
                do_group(2 * j + 1, rows1, ssem1)
                return 0

            lax.fori_loop(1, n_pairs, pair_body, 0)
        if leftover:
            if n_pairs > 0:
                drain_store(ssem0)
            do_group(n_groups - 1, rows0, ssem0)
        drain_store(ssem0)
        if n_pairs > 0:
            drain_store(ssem1)

    return functools.partial(
        pl.kernel,
        out_type=jax.ShapeDtypeStruct((n_idx, _BASIS), jnp.float32),
        mesh=mesh,
        scratch_types=[
            pltpu.VMEM((per_w,), jnp.int32),
            pltpu.VMEM((gsz, _BASIS), jnp.float32),
            pltpu.VMEM((gsz, _BASIS), jnp.float32),
            pltpu.SemaphoreType.DMA,
            pltpu.SemaphoreType.DMA,
            pltpu.SemaphoreType.DMA,
        ],
    )(body)


def _make_sc_scatter_v1(group):
    per_w = _E // _NW
    gsz = _CH * group
    n_groups = per_w // gsz
    rows_per_tile = _NPAD // _NS
    mesh = plsc.VectorSubcoreMesh(core_axis_name="c", subcore_axis_name="s", num_cores=_NC, num_subcores=_NS)

    def body(m_hbm, dst3_hbm, out_hbm, idx_v, rows_v, acc_sh, sem, sem2):
        cid = lax.axis_index("c")
        sid = lax.axis_index("s")
        wid = sid * _NC + cid
        sems = (sem, sem2)

        zeros16 = jnp.zeros((16,), jnp.float32)

        def zrow(r, _):
            for cc in range(_BASIS // 16):
                rows_v[0, r, pl.ds(cc * 16, 16)] = zeros16
            return 0

        lax.fori_loop(0, _CH, zrow, 0)
        for k in range(rows_per_tile // _CH):
            pltpu.sync_copy(
                rows_v.at[0],
                acc_sh.at[pl.ds(sid * rows_per_tile + k * _CH, _CH)],
            )
        plsc.subcore_barrier()

        def group_body(g, _):
            base = wid * per_w + g * gsz
            brow = base // gsz
            pltpu.sync_copy(dst3_hbm.at[brow], idx_v)
            cp0 = pltpu.async_copy(
                m_hbm.at[pl.ds(base, _CH)], rows_v.at[0], sems[0])
            copies = [cp0]
            for k in range(group):
                if k + 1 < group:
                    copies.append(pltpu.async_copy(
                        m_hbm.at[pl.ds(base + (k + 1) * _CH, _CH)],
                        rows_v.at[(k + 1) % 2], sems[(k + 1) % 2]))
                copies[k].wait()
                pltpu.sync_copy(
                    rows_v.at[k % 2],
                    acc_sh.at[idx_v.at[k]],
                    add=True,
                )
            return 0

        lax.fori_loop(0, n_groups, group_body, 0)
        plsc.subcore_barrier()
        pltpu.sync_copy(
            acc_sh.at[pl.ds(sid * rows_per_tile, rows_per_tile)],
            out_hbm.at[pl.ds(cid * _NPAD + sid * rows_per_tile, rows_per_tile)],
        )

    return functools.partial(
        pl.kernel,
        out_type=jax.ShapeDtypeStruct((_NC * _NPAD, _BASIS), jnp.float32),
        mesh=mesh,
        scratch_types=[
            pltpu.VMEM((group, _CH), jnp.int32),
            pltpu.VMEM((2, _CH, _BASIS), jnp.float32),
            pltpu.VMEM_SHARED((_NPAD, _BASIS), jnp.float32),
            pltpu.SemaphoreType.DMA,
            pltpu.SemaphoreType.DMA,
        ],
    )(body)


_SCH = 40
_SGRP = 10


def _make_sc_scatter():
    per_w = _E // _NW
    gsz = _SCH * _SGRP
    n_groups = per_w // gsz
    assert per_w % gsz == 0
    rows_per_tile = _NPAD // _NS
    mesh = plsc.VectorSubcoreMesh(core_axis_name="c", subcore_axis_name="s", num_cores=_NC, num_subcores=_NS)

    n_pairs = n_groups // 2
    leftover = n_groups % 2 == 1

    def body(m_hbm, dst4_hbm, out_hbm, idx_a, idx_b, rows_v, acc_sh,
             lsem0, lsem1, scsem, isem0, isem1):
        cid = lax.axis_index("c")
        sid = lax.axis_index("s")
        wid = sid * _NC + cid
        lsems = (lsem0, lsem1)
        isems = (isem0, isem1)
        idxs = (idx_a, idx_b)
        ebase = wid * per_w
        ibase = wid * n_groups

        def load_idx(g, slot):
            pltpu.async_copy(dst4_hbm.at[ibase + g], idxs[slot],
                             isems[slot])

        def drain_idx(slot):
            pltpu.make_async_copy(
                dst4_hbm.at[0], idxs[slot], isems[slot]).wait()

        load_idx(0, 0)

        zeros16 = jnp.zeros((16,), jnp.float32)

        def zrow(r, _):
            for cc in range(_BASIS // 16):
                rows_v[0, r, pl.ds(cc * 16, 16)] = zeros16
            return 0

        lax.fori_loop(0, _SCH, zrow, 0)
        zcopies = [
            pltpu.async_copy(
                rows_v.at[0],
                acc_sh.at[pl.ds(sid * rows_per_tile + z * _SCH, _SCH)],
                scsem,
            )
            for z in range(rows_per_tile // _SCH)
        ]
        for zc in zcopies:
            zc.wait()
        plsc.subcore_barrier()

        def load(g, c, buf):
            return pltpu.async_copy(
                m_hbm.at[pl.ds(ebase + g * gsz + c * _SCH, _SCH)],
                rows_v.at[buf], lsems[buf])

        def drain_scatter():
            pltpu.make_async_copy(
                m_hbm.at[pl.ds(0, _SCH)], acc_sh.at[pl.ds(0, _SCH)],
                scsem).wait()

        def drain_load(buf):
            pltpu.make_async_copy(
                m_hbm.at[pl.ds(0, _SCH)], rows_v.at[buf], lsems[buf]).wait()

        load(0, 0, 0)

        def process_group(g, islot):
            for c in range(_SGRP):
                b = c % 2
                nb = (c + 1) % 2
                if c > 0:
                    drain_scatter()
                if c + 1 < _SGRP:
                    load(g, c + 1, nb)
                else:
                    g_next = lax.rem(g + 1, n_groups)
                    load(g_next, 0, nb)
                drain_load(b)
                pltpu.async_copy(
                    rows_v.at[b], acc_sh.at[idxs[islot].at[c]], scsem,
                    add=True)
            drain_scatter()

        def pair_body(j, _):
            load_idx(2 * j + 1, 1)
            drain_idx(0)
            process_group(2 * j, 0)
            load_idx(2 * j + 2, 0)
            drain_idx(1)
            process_group(2 * j + 1, 1)
            return 0

        lax.fori_loop(0, n_pairs, pair_body, 0)
        if leftover:
            drain_idx(0)
            process_group(n_groups - 1, 0)
        drain_load(0)
        plsc.subcore_barrier()
        pltpu.sync_copy(
            acc_sh.at[pl.ds(sid * rows_per_tile, rows_per_tile)],
            out_hbm.at[pl.ds(cid * _NPAD + sid * rows_per_tile, rows_per_tile)],
        )

    return functools.partial(
        pl.kernel,
        out_type=jax.ShapeDtypeStruct((_NC * _NPAD, _BASIS), jnp.float32),
        mesh=mesh,
        scratch_types=[
            pltpu.VMEM((_SGRP, _SCH), jnp.int32),
            pltpu.VMEM((_SGRP, _SCH), jnp.int32),
            pltpu.VMEM((2, _SCH, _BASIS), jnp.float32),
            pltpu.VMEM_SHARED((_NPAD, _BASIS), jnp.float32),
            pltpu.SemaphoreType.DMA,
            pltpu.SemaphoreType.DMA,
            pltpu.SemaphoreType.DMA,
            pltpu.SemaphoreType.DMA,
            pltpu.SemaphoreType.DMA,
        ],
    )(body)


_sc_gather_nodes = _make_sc_gather(10240, ch=40, group=8)
_sc_gather_edges = _make_sc_gather(_E, ch=40, group=5)
_sc_scatter = _make_sc_scatter()
_sc_scatter_v1 = _make_sc_scatter_v1(group=5)


_BN = 2000
_BE = 2000


def _full(shape):
    return pl.BlockSpec(shape, lambda i: (0,) * len(shape))


def _ccf_first_body(c_ref, w_ref, b_ref, ccf_ref):
    ccf_ref[...] = (
        jnp.dot(c_ref[...], w_ref[...], preferred_element_type=jnp.float32)
        + b_ref[...]
    )


def _ccf_first(C, cf_w, cf_b):
    return pl.pallas_call(
        _ccf_first_body,
        grid=(_N // _BN,),
        in_specs=[
            pl.BlockSpec((_BN, _BASIS), lambda i: (i, 0)),
            _full((_BASIS, _BASIS)),
            _full((1, _BASIS)),
        ],
        out_specs=pl.BlockSpec((_BN, _BASIS), lambda i: (i, 0)),
        out_shape=jax.ShapeDtypeStruct((_N, _BASIS), jnp.float32),
    )(C, cf_w, cf_b.reshape(1, _BASIS))


def _ccf_step_body(c_ref, a0_ref, a1_ref, w_ref, b_ref, cn_ref, ccf_ref):
    c = c_ref[...] + a0_ref[...] + a1_ref[...]
    cn_ref[...] = c
    ccf_ref[...] = (
        jnp.dot(c, w_ref[...], preferred_element_type=jnp.float32) + b_ref[...]
    )


def _ccf_step(C, a0, a1, cf_w, cf_b):
    blk = pl.BlockSpec((_BN, _BASIS), lambda i: (i, 0))
    nb = _N // _BN
    return pl.pallas_call(
        _ccf_step_body,
        grid=(nb,),
        in_specs=[
            blk,
            blk,
            blk,
            _full((_BASIS, _BASIS)),
            _full((1, _BASIS)),
        ],
        out_specs=[blk, blk],
        out_shape=[
            jax.ShapeDtypeStruct((_N, _BASIS), jnp.float32),
            jax.ShapeDtypeStruct((_N, _BASIS), jnp.float32),
        ],
    )(C, a0, a1, cf_w, cf_b.reshape(1, _BASIS))


def _dfe_body(e_ref, w_ref, b_ref, o_ref):
    o_ref[...] = (
        jnp.dot(e_ref[...], w_ref[...], preferred_element_type=jnp.float32)
        + b_ref[...]
    )


def _dfe(edge_attr, df_w, df_b):
    return pl.pallas_call(
        _dfe_body,
        grid=(_E // _BE,),
        in_specs=[
            pl.BlockSpec((_BE, _NG), lambda i: (i, 0)),
            _full((_NG, _BASIS)),
            _full((1, _BASIS)),
        ],
        out_specs=pl.BlockSpec((_BE, _BASIS), lambda i: (i, 0)),
        out_shape=jax.ShapeDtypeStruct((_E, _BASIS), jnp.float32),
    )(edge_attr, df_w, df_b.reshape(1, _BASIS))


def _edge_mm_body(g_ref, d_ref, w_ref, m_ref):
    p = g_ref[...] * d_ref[...]
    m_ref[...] = jnp.tanh(
        jnp.dot(p, w_ref[...], preferred_element_type=jnp.float32)
    )


def _edge_mm(G, dfe, fc_w):
    blk = pl.BlockSpec((_BE, _BASIS), lambda i: (i, 0))
    return pl.pallas_call(
        _edge_mm_body,
        grid=(_E // _BE,),
        in_specs=[blk, blk, _full((_BASIS, _BASIS))],
        out_specs=blk,
        out_shape=jax.ShapeDtypeStruct((_E, _BASIS), jnp.float32),
    )(G, dfe, fc_w)


def _readout_body(c_ref, a0_ref, a1_ref, b_ref, w1_ref, b1_ref, w2_ref,
                  b2_ref, o_ref):
    i = pl.program_id(0)
    c = c_ref[...] + a0_ref[...] + a1_ref[...]
    h1 = jnp.tanh(
        jnp.dot(c, w1_ref[...], preferred_element_type=jnp.float32)
        + b1_ref[...]
    )
    h = jnp.dot(h1, w2_ref[...], preferred_element_type=jnp.float32) + b2_ref[...]
    ids = b_ref[0, 0, :]
    onehot = (
        ids[:, None] == lax.broadcasted_iota(jnp.int32, (_BN, _NGRAPHS), 1)
    ).astype(jnp.float32)
    pooled = lax.dot_general(
        onehot, h, (((0,), (0,)), ((), ())),
        preferred_element_type=jnp.float32,
    )

    @pl.when(i == 0)
    def _():
        o_ref[...] = jnp.zeros_like(o_ref)

    o_ref[...] += pooled


def _readout(C, a0, a1, batch, mlp_w1, mlp_b1, mlp_w2, mlp_b2):
    nb = _N // _BN
    blk = pl.BlockSpec((_BN, _BASIS), lambda i: (i, 0))
    return pl.pallas_call(
        _readout_body,
        grid=(nb,),
        in_specs=[
            blk,
            blk,
            blk,
            pl.BlockSpec((1, 1, _BN), lambda i: (i, 0, 0)),
            _full((_BASIS, _HID)),
            _full((1, _HID)),
            _full((_HID, 4)),
            _full((1, 4)),
        ],
        out_specs=pl.BlockSpec((_NGRAPHS, 4), lambda i: (0, 0)),
        out_shape=jax.ShapeDtypeStruct((_NGRAPHS, 4), jnp.float32),
    )(C, a0, a1, batch.reshape(nb, 1, _BN), mlp_w1,
      mlp_b1.reshape(1, _HID), mlp_w2, mlp_b2.reshape(1, 4))


def kernel(Z, edge_index, edge_attr, batch, embed, cf_w, cf_b, df_w, df_b,
           fc_w, mlp_w1, mlp_b1, mlp_w2, mlp_b2):
    src = edge_index[0].astype(jnp.int32)
    dst = edge_index[1].astype(jnp.int32)
    Zp = jnp.concatenate([Z.astype(jnp.int32), jnp.zeros((240,), jnp.int32)])
    dst4 = dst.reshape(-1, _SGRP, _SCH)
    dst3 = dst.reshape(-1, 5, _CH)

    C = _sc_gather_nodes(embed, Zp)[:_N]
    dfe = _dfe(edge_attr, df_w, df_b)

    a0 = a1 = None
    for t in range(_T):
        if t == 0:
            Ccf = _ccf_first(C, cf_w, cf_b)
        else:
            C, Ccf = _ccf_step(C, a0, a1, cf_w, cf_b)
        G = _sc_gather_edges(Ccf, src)
        M = _edge_mm(G, dfe, fc_w)
        agg = _sc_scatter_v1(M, dst3)
        a0 = agg[:_N]
        a1 = agg[_NPAD:_NPAD + _N]

    return _readout(C, a0, a1, batch, mlp_w1, mlp_b1, mlp_w2, mlp_b2)


# speedup vs baseline: 2.9080x; 1.0327x over previous
import functools

import jax
import jax.numpy as jnp
from jax import lax
from jax.experimental import pallas as pl
from jax.experimental.pallas import tpu as pltpu
from jax.experimental.pallas import tpu_sc as plsc

_N = 10000
_E = 320000
_BASIS = 128
_NG = 16
_HID = 256
_NGRAPHS = 64
_T = 3
_NPAD = 10240

_NC = 2
_NS = 16
_NW = _NC * _NS

_CH = 80


def _make_sc_gather(n_idx, ch, group, ncols=_BASIS, dtype=jnp.float32):
    per_w = n_idx // _NW
    gsz = ch * group
    n_groups = per_w // gsz
    assert per_w % gsz == 0
    mesh = plsc.VectorSubcoreMesh(core_axis_name="c", subcore_axis_name="s", num_cores=_NC, num_subcores=_NS)

    n_pairs = n_groups // 2
    leftover = n_groups % 2 == 1

    def body(table_hbm, idx_hbm, out_hbm, idx_full, rows0, rows1,
             gsem, ssem0, ssem1):
        wid = lax.axis_index("s") * _NC + lax.axis_index("c")
        base0 = wid * per_w
        pltpu.sync_copy(idx_hbm.at[pl.ds(base0, per_w)], idx_full)

        def drain_store(ssem):
            pltpu.make_async_copy(
                rows0, out_hbm.at[pl.ds(0, gsz)], ssem).wait()

        def do_group(g, rows, ssem):
            off = g * gsz
            copies = [
                pltpu.async_copy(
                    table_hbm.at[idx_full.at[pl.ds(off + k * ch, ch)]],
                    rows.at[pl.ds(k * ch, ch)],
                    gsem,
                )
                for k in range(group)
            ]
            for c in copies:
                c.wait()
            pltpu.async_copy(rows, out_hbm.at[pl.ds(base0 + off, gsz)], ssem)

        if n_pairs > 0:
            do_group(0, rows0, ssem0)
            do_group(1, rows1, ssem1)

            def pair_body(j, _):
                drain_store(ssem0)
                do_group(2 * j, rows0, ssem0)
                drain_store(ssem1)
                do_group(2 * j + 1, rows1, ssem1)
                return 0

            lax.fori_loop(1, n_pairs, pair_body, 0)
        if leftover:
            if n_pairs > 0:
                drain_store(ssem0)
            do_group(n_groups - 1, rows0, ssem0)
        drain_store(ssem0)
        if n_pairs > 0:
            drain_store(ssem1)

    return functools.partial(
        pl.kernel,
        out_type=jax.ShapeDtypeStruct((n_idx, ncols), dtype),
        mesh=mesh,
        scratch_types=[
            pltpu.VMEM((per_w,), jnp.int32),
            pltpu.VMEM((gsz, ncols), dtype),
            pltpu.VMEM((gsz, ncols), dtype),
            pltpu.SemaphoreType.DMA,
            pltpu.SemaphoreType.DMA,
            pltpu.SemaphoreType.DMA,
        ],
    )(body)


def _make_sc_scatter_v1(group):
    per_w = _E // _NW
    gsz = _CH * group
    n_groups = per_w // gsz
    rows_per_tile = _NPAD // _NS
    mesh = plsc.VectorSubcoreMesh(core_axis_name="c", subcore_axis_name="s", num_cores=_NC, num_subcores=_NS)

    def body(m_hbm, dst3_hbm, out_hbm, idx_v, rows_v, acc_sh, sem, sem2,
             scsem):
        cid = lax.axis_index("c")
        sid = lax.axis_index("s")
        wid = sid * _NC + cid
        sems = (sem, sem2)

        zeros16 = jnp.zeros((16,), jnp.float32)
        zeros16i = jnp.zeros((16,), jnp.int32)

        def zrow(r, _):
            for cc in range(_BASIS // 16):
                rows_v[0, r, pl.ds(cc * 16, 16)] = zeros16
            return 0

        lax.fori_loop(0, _CH, zrow, 0)
        for k in range(rows_per_tile // _CH):
            pltpu.sync_copy(
                rows_v.at[0],
                acc_sh.at[pl.ds(sid * rows_per_tile + k * _CH, _CH)],
            )
        plsc.subcore_barrier()

        def group_body(g, _):
            base = wid * per_w + g * gsz
            brow = base // gsz
            pltpu.sync_copy(dst3_hbm.at[brow], idx_v)
            cp0 = pltpu.async_copy(
                m_hbm.at[pl.ds(base, _CH)], rows_v.at[0], sems[0])
            copies = [cp0]
            scat = None
            for k in range(group):
                if scat is not None:
                    scat.wait()
                if k + 1 < group:
                    copies.append(pltpu.async_copy(
                        m_hbm.at[pl.ds(base + (k + 1) * _CH, _CH)],
                        rows_v.at[(k + 1) % 2], sems[(k + 1) % 2]))
                copies[k].wait()
                scat = pltpu.async_copy(
                    rows_v.at[k % 2],
                    acc_sh.at[idx_v.at[k]],
                    scsem,
                    add=True,
                )
            scat.wait()
            return 0

        lax.fori_loop(0, n_groups, group_body, 0)
        plsc.subcore_barrier()
        pltpu.sync_copy(
            acc_sh.at[pl.ds(sid * rows_per_tile, rows_per_tile)],
            out_hbm.at[pl.ds(cid * _NPAD + sid * rows_per_tile, rows_per_tile)],
        )

    return functools.partial(
        pl.kernel,
        out_type=jax.ShapeDtypeStruct((_NC * _NPAD, _BASIS), jnp.float32),
        mesh=mesh,
        scratch_types=[
            pltpu.VMEM((group, _CH), jnp.int32),
            pltpu.VMEM((2, _CH, _BASIS), jnp.float32),
            pltpu.VMEM_SHARED((_NPAD, _BASIS), jnp.float32),
            pltpu.SemaphoreType.DMA,
            pltpu.SemaphoreType.DMA,
            pltpu.SemaphoreType.DMA,
        ],
    )(body)


_sc_gather_nodes = _make_sc_gather(10240, ch=40, group=8)
_sc_gather_edges = _make_sc_gather(_E, ch=40, group=5)
_sc_scatter_v1 = _make_sc_scatter_v1(group=5)


_BN = 2000
_BE = 2000


def _full(shape):
    return pl.BlockSpec(shape, lambda i: (0,) * len(shape))


def _ccf_first_body(c_ref, w_ref, b_ref, ccf_ref):
    c = c_ref[...]
    ccf_ref[...] = (
        jnp.dot(c, w_ref[...], preferred_element_type=jnp.float32)
        + b_ref[...]
    )


def _ccf_first(C, cf_w, cf_b):
    return pl.pallas_call(
        _ccf_first_body,
        grid=(_N // _BN,),
        in_specs=[
            pl.BlockSpec((_BN, _BASIS), lambda i: (i, 0)),
            _full((_BASIS, _BASIS)),
            _full((1, _BASIS)),
        ],
        out_specs=pl.BlockSpec((_BN, _BASIS), lambda i: (i, 0)),
        out_shape=jax.ShapeDtypeStruct((_N, _BASIS), jnp.float32),
    )(C, cf_w, cf_b.reshape(1, _BASIS))


def _ccf_step_body(c_ref, a0_ref, a1_ref, w_ref, b_ref, cn_ref, ccf_ref):
    c = c_ref[...] + a0_ref[...] + a1_ref[...]
    cn_ref[...] = c
    ccf_ref[...] = (
        jnp.dot(c, w_ref[...], preferred_element_type=jnp.float32) + b_ref[...]
    )


def _ccf_step(C, a0, a1, cf_w, cf_b):
    blk = pl.BlockSpec((_BN, _BASIS), lambda i: (i, 0))
    nb = _N // _BN
    return pl.pallas_call(
        _ccf_step_body,
        grid=(nb,),
        in_specs=[
            blk,
            blk,
            blk,
            _full((_BASIS, _BASIS)),
            _full((1, _BASIS)),
        ],
        out_specs=[blk, blk],
        out_shape=[
            jax.ShapeDtypeStruct((_N, _BASIS), jnp.float32),
            jax.ShapeDtypeStruct((_N, _BASIS), jnp.float32),
        ],
    )(C, a0, a1, cf_w, cf_b.reshape(1, _BASIS))


def _dfe_body(e_ref, w_ref, b_ref, o_ref):
    o_ref[...] = (
        jnp.dot(e_ref[...], w_ref[...], preferred_element_type=jnp.float32)
        + b_ref[...]
    ).astype(jnp.bfloat16)


def _dfe(edge_attr, df_w, df_b):
    return pl.pallas_call(
        _dfe_body,
        grid=(_E // _BE,),
        in_specs=[
            pl.BlockSpec((_BE, _NG), lambda i: (i, 0)),
            _full((_NG, _BASIS)),
            _full((1, _BASIS)),
        ],
        out_specs=pl.BlockSpec((_BE, _BASIS), lambda i: (i, 0)),
        out_shape=jax.ShapeDtypeStruct((_E, _BASIS), jnp.bfloat16),
    )(edge_attr, df_w, df_b.reshape(1, _BASIS))


def _edge_mm_body(g_ref, d_ref, w_ref, m_ref):
    p = g_ref[...].astype(jnp.float32) * d_ref[...].astype(jnp.float32)
    m_ref[...] = jnp.tanh(
        jnp.dot(p, w_ref[...], preferred_element_type=jnp.float32)
    )


def _edge_mm(G, dfe, fc_w):
    blk = pl.BlockSpec((_BE, _BASIS), lambda i: (i, 0))
    return pl.pallas_call(
        _edge_mm_body,
        grid=(_E // _BE,),
        in_specs=[blk, blk, _full((_BASIS, _BASIS))],
        out_specs=pl.BlockSpec((_BE, _BASIS), lambda i: (i, 0)),
        out_shape=jax.ShapeDtypeStruct((_E, _BASIS), jnp.float32),
    )(G, dfe, fc_w)


def _readout_body(c_ref, a0_ref, a1_ref, b_ref, w1_ref, b1_ref, w2_ref,
                  b2_ref, o_ref):
    i = pl.program_id(0)
    c = c_ref[...] + a0_ref[...] + a1_ref[...]
    h1 = jnp.tanh(
        jnp.dot(c, w1_ref[...], preferred_element_type=jnp.float32)
        + b1_ref[...]
    )
    h = jnp.dot(h1, w2_ref[...], preferred_element_type=jnp.float32) + b2_ref[...]
    ids = b_ref[0, 0, :]
    onehot = (
        ids[:, None] == lax.broadcasted_iota(jnp.int32, (_BN, _NGRAPHS), 1)
    ).astype(jnp.float32)
    pooled = lax.dot_general(
        onehot, h, (((0,), (0,)), ((), ())),
        preferred_element_type=jnp.float32,
    )

    @pl.when(i == 0)
    def _():
        o_ref[...] = jnp.zeros_like(o_ref)

    o_ref[...] += pooled


def _readout(C, a0, a1, batch, mlp_w1, mlp_b1, mlp_w2, mlp_b2):
    nb = _N // _BN
    blk = pl.BlockSpec((_BN, _BASIS), lambda i: (i, 0))
    return pl.pallas_call(
        _readout_body,
        grid=(nb,),
        in_specs=[
            blk,
            blk,
            blk,
            pl.BlockSpec((1, 1, _BN), lambda i: (i, 0, 0)),
            _full((_BASIS, _HID)),
            _full((1, _HID)),
            _full((_HID, 4)),
            _full((1, 4)),
        ],
        out_specs=pl.BlockSpec((_NGRAPHS, 4), lambda i: (0, 0)),
        out_shape=jax.ShapeDtypeStruct((_NGRAPHS, 4), jnp.float32),
    )(C, a0, a1, batch.reshape(nb, 1, _BN), mlp_w1,
      mlp_b1.reshape(1, _HID), mlp_w2, mlp_b2.reshape(1, 4))


def kernel(Z, edge_index, edge_attr, batch, embed, cf_w, cf_b, df_w, df_b,
           fc_w, mlp_w1, mlp_b1, mlp_w2, mlp_b2):
    src = edge_index[0].astype(jnp.int32)
    dst = edge_index[1].astype(jnp.int32)
    Zp = jnp.concatenate([Z.astype(jnp.int32), jnp.zeros((240,), jnp.int32)])
    dst3 = dst.reshape(-1, 5, _CH)

    C = _sc_gather_nodes(embed, Zp)[:_N]
    dfe = _dfe(edge_attr, df_w, df_b)

    a0 = a1 = None
    for t in range(_T):
        if t == 0:
            Ccf = _ccf_first(C, cf_w, cf_b)
        else:
            C, Ccf = _ccf_step(C, a0, a1, cf_w, cf_b)
        G = _sc_gather_edges(Ccf, src)
        M = _edge_mm(G, dfe, fc_w)
        agg = _sc_scatter_v1(M, dst3)
        a0 = agg[:_N]
        a1 = agg[_NPAD:_NPAD + _N]

    return _readout(C, a0, a1, batch, mlp_w1, mlp_b1, mlp_w2, mlp_b2)
